# baseline (speedup 1.0000x reference)
import functools

import jax
import jax.numpy as jnp
from jax import lax
from jax.experimental import pallas as pl
from jax.experimental.pallas import tpu as pltpu
from jax.experimental.pallas import tpu_sc as plsc

_EPS = 1e-6


def _gather16(x, perm, L):
    dnums = lax.GatherDimensionNumbers(
        offset_dims=(), collapsed_slice_dims=(0,), start_index_map=(0,)
    )
    return lax.gather(
        x, perm.reshape(L, 1), dimension_numbers=dnums, slice_sizes=(1,),
        mode=lax.GatherScatterMode.PROMISE_IN_BOUNDS,
    )


def _vrsqrt(x):
    i = lax.bitcast_convert_type(x, jnp.int32)
    i = jnp.int32(0x5F3759DF) - lax.shift_right_arithmetic(i, 1)
    y = lax.bitcast_convert_type(i, jnp.float32)
    for _ in range(3):
        y = y * (1.5 - 0.5 * x * y * y)
    return y


@functools.cache
def _make_fused(V, D, B):
    info = plsc.get_sparse_core_info()
    NC, NS, L = info.num_cores, info.num_subcores, info.num_lanes
    NW = NC * NS
    n_vreg = D // L
    b_per_w = B // NW
    C = 64
    NG_BUF = 3
    NO_BUF = 2
    NG = C // L
    n_chunks = b_per_w // C
    assert B % (8 * NW) == 0 and D % L == 0 and C % L == 0 and n_chunks >= 4

    mesh = plsc.VectorSubcoreMesh(core_axis_name="c", subcore_axis_name="s")

    @functools.partial(
        pl.kernel,
        mesh=mesh,
        out_type=jax.ShapeDtypeStruct((B, D), jnp.float32),
        scratch_types=[
            pltpu.VMEM((b_per_w,), jnp.int32),
            pltpu.VMEM((NG_BUF * C, D), jnp.float32),
            pltpu.VMEM((NO_BUF * C, D), jnp.float32),
            pltpu.VMEM((D,), jnp.float32),
            pltpu.SemaphoreType.DMA((NG_BUF,)),
            pltpu.SemaphoreType.DMA((NO_BUF,)),
        ],
    )
    def k(ids_hbm, table_hbm, w_hbm, out_hbm, idx_v, gbuf, obuf, wv,
          gsem, ssem):
        wid = lax.axis_index("s") * NC + lax.axis_index("c")
        base = wid * b_per_w
        iota = jnp.arange(L, dtype=jnp.int32)
        masks = [jnp.asarray((iota & m) != 0) for m in (1, 2, 4, 8)]
        perms = [jnp.asarray(iota ^ m, dtype=jnp.int32) for m in (1, 2, 4, 8)]

        pltpu.sync_copy(w_hbm, wv)
        pltpu.sync_copy(ids_hbm.at[pl.ds(base, b_per_w)], idx_v)
        ws = [wv[pl.ds(j * L, L)] for j in range(n_vreg)]

        def gather(c, slot):
            pltpu.async_copy(
                table_hbm.at[idx_v.at[pl.ds(c * C, C)]],
                gbuf.at[pl.ds(slot * C, C)], gsem.at[slot],
            )

        def wait_gather(slot):
            pltpu.make_async_copy(
                table_hbm.at[pl.ds(0, C)], gbuf.at[pl.ds(slot * C, C)],
                gsem.at[slot],
            ).wait()

        def store(c, slot):
            pltpu.async_copy(
                obuf.at[pl.ds(slot * C, C)],
                out_hbm.at[pl.ds(base + c * C, C)], ssem.at[slot],
            )

        def wait_store(slot):
            pltpu.make_async_copy(
                obuf.at[pl.ds(slot * C, C)], out_hbm.at[pl.ds(base, C)],
                ssem.at[slot],
            ).wait()

        def combine(x, y, lvl):
            m = masks[lvl]
            return jnp.where(m, y, x) + _gather16(
                jnp.where(m, x, y), perms[lvl], L
            )

        def sq_group(rbase):
            partial = []
            for i in range(L):
                a0 = jnp.zeros((L,), jnp.float32)
                a1 = jnp.zeros((L,), jnp.float32)
                a2 = jnp.zeros((L,), jnp.float32)
                for j in range(0, n_vreg, 3):
                    v = gbuf[rbase + i, pl.ds(j * L, L)]
                    a0 = a0 + v * v
                    v = gbuf[rbase + i, pl.ds((j + 1) * L, L)]
                    a1 = a1 + v * v
                    v = gbuf[rbase + i, pl.ds((j + 2) * L, L)]
                    a2 = a2 + v * v
                node, lvl = a0 + (a1 + a2), 0
                while partial and partial[-1][0] == lvl:
                    node = combine(partial.pop()[1], node, lvl)
                    lvl += 1
                partial.append((lvl, node))
            return _vrsqrt(partial[0][1] * (1.0 / D) + _EPS)

        def rescale_group(src_base, dst_base, inv):
            for i in range(L):
                sb = _gather16(inv, iota * 0 + i, L)
                for j in range(n_vreg):
                    obuf[dst_base + i, pl.ds(j * L, L)] = (
                        gbuf[src_base + i, pl.ds(j * L, L)] * (ws[j] * sb)
                    )

        def chunk_body(c, inv_carry):
            gslot = lax.rem(c, NG_BUF)
            oslot = lax.rem(c, NO_BUF)
            pgslot = jnp.where(c > 0, lax.rem(c - 1, NG_BUF), NG_BUF - 1)
            poslot = jnp.where(c > 0, lax.rem(c - 1, NO_BUF), NO_BUF - 1)

            @pl.when(c < n_chunks)
            def _():
                wait_gather(gslot)

            @pl.when(c >= NO_BUF)
            def _():
                wait_store(oslot)

            def gbody(g, inv_prev):
                src = jnp.where(g > 0, gslot * C + (g - 1) * L,
                                pgslot * C + (NG - 1) * L)
                dst = jnp.where(g > 0, oslot * C + (g - 1) * L,
                                poslot * C + (NG - 1) * L)
                rescale_group(src, dst, inv_prev)
                return sq_group(gslot * C + g * L)

            inv_out = lax.fori_loop(0, NG, gbody, inv_carry)

            @pl.when(c > 0)
            def _():
                store(c - 1, poslot)

            @pl.when(c + 2 < n_chunks)
            def _():
                gather(c + 2, lax.rem(c + 2, NG_BUF))

            return inv_out

        gather(0, 0)
        gather(1, 1)
        lax.fori_loop(0, n_chunks + 1, chunk_body, jnp.zeros((L,), jnp.float32))
        wait_store(lax.rem(jnp.int32(n_chunks - 1), NO_BUF))

    return k


def kernel(input_ids, tok_embeddings, norm_weight):
    Bt, S = input_ids.shape
    V, D = tok_embeddings.shape
    ids = input_ids.reshape(-1)
    out = _make_fused(V, D, Bt * S)(ids, tok_embeddings, norm_weight)
    return out.reshape(Bt, S, D)


# speedup vs baseline: 5.4786x; 5.4786x over previous
import functools

import jax
import jax.numpy as jnp
from jax import lax
from jax.experimental import pallas as pl
from jax.experimental.pallas import tpu as pltpu
from jax.experimental.pallas import tpu_sc as plsc

_EPS = 1e-6


def _gather16(x, perm, L):
    dnums = lax.GatherDimensionNumbers(
        offset_dims=(), collapsed_slice_dims=(0,), start_index_map=(0,)
    )
    return lax.gather(
        x, perm.reshape(L, 1), dimension_numbers=dnums, slice_sizes=(1,),
        mode=lax.GatherScatterMode.PROMISE_IN_BOUNDS,
    )


def _vrsqrt(x):
    i = lax.bitcast_convert_type(x, jnp.int32)
    i = jnp.int32(0x5F3759DF) - lax.shift_right_arithmetic(i, 1)
    y = lax.bitcast_convert_type(i, jnp.float32)
    for _ in range(3):
        y = y * (1.5 - 0.5 * x * y * y)
    return y


@functools.cache
def _make_fused(V, D, B):
    info = plsc.get_sparse_core_info()
    NC, NS, L = info.num_cores, info.num_subcores, info.num_lanes
    NW = NC * NS
    n_vreg = D // L
    b_per_w = B // NW
    C = 64
    NBUF = 4
    n_chunks = b_per_w // C
    assert B % (8 * NW) == 0 and D % L == 0 and C % L == 0 and n_chunks >= 4

    mesh = plsc.VectorSubcoreMesh(core_axis_name="c", subcore_axis_name="s")

    @functools.partial(
        pl.kernel,
        mesh=mesh,
        out_type=jax.ShapeDtypeStruct((B, D), jnp.float32),
        scratch_types=[
            pltpu.VMEM((b_per_w,), jnp.int32),
            pltpu.VMEM((NBUF * C, D), jnp.float32),
            pltpu.VMEM((D,), jnp.float32),
            pltpu.VMEM((C, 16), jnp.float32),
            pltpu.SemaphoreType.DMA((NBUF,)),
            pltpu.SemaphoreType.DMA((NBUF,)),
        ],
    )
    def k(ids_hbm, table_hbm, w_hbm, out_hbm, idx_v, buf, wv, sc_v, gsem, ssem):
        wid = lax.axis_index("s") * NC + lax.axis_index("c")
        base = wid * b_per_w
        iota = jnp.arange(L, dtype=jnp.int32)
        masks = [jnp.asarray((iota & m) != 0) for m in (1, 2, 4, 8)]
        perms = [jnp.asarray(iota ^ m, dtype=jnp.int32) for m in (1, 2, 4, 8)]

        pltpu.sync_copy(w_hbm, wv)
        pltpu.sync_copy(ids_hbm.at[pl.ds(base, b_per_w)], idx_v)
        ws = [wv[pl.ds(j * L, L)] for j in range(n_vreg)]

        def bslice(slot):
            return buf.at[pl.ds(slot * C, C)]

        def gather(c, slot):
            pltpu.async_copy(
                table_hbm.at[idx_v.at[pl.ds(c * C, C)]], bslice(slot),
                gsem.at[slot],
            )

        def wait_gather(slot):
            pltpu.make_async_copy(
                table_hbm.at[pl.ds(0, C)], bslice(slot), gsem.at[slot]
            ).wait()

        def store(c, slot):
            pltpu.async_copy(
                bslice(slot), out_hbm.at[pl.ds(base + c * C, C)], ssem.at[slot]
            )

        def wait_store(slot):
            pltpu.make_async_copy(
                bslice(slot), out_hbm.at[pl.ds(base, C)], ssem.at[slot]
            ).wait()

        def combine(x, y, lvl):
            m = masks[lvl]
            return jnp.where(m, y, x) + _gather16(
                jnp.where(m, x, y), perms[lvl], L
            )

        def normalize_chunk(slot):
            row0 = slot * C

            @plsc.parallel_loop(0, C, unroll=2)
            def _(r):
                a0 = jnp.zeros((L,), jnp.float32)
                a1 = jnp.zeros((L,), jnp.float32)
                a2 = jnp.zeros((L,), jnp.float32)
                for j in range(0, n_vreg, 3):
                    v = buf[row0 + r, pl.ds(j * L, L)]
                    a0 = a0 + v * v
                    v = buf[row0 + r, pl.ds((j + 1) * L, L)]
                    a1 = a1 + v * v
                    v = buf[row0 + r, pl.ds((j + 2) * L, L)]
                    a2 = a2 + v * v
                acc = a0 + (a1 + a2)
                for lvl in range(4):
                    acc = acc + _gather16(acc, perms[lvl], L)
                sc_v[r, pl.ds(0, L)] = _vrsqrt(acc * (1.0 / D) + _EPS)

            @plsc.parallel_loop(0, C, unroll=2)
            def _(r):
                sb = sc_v[r, pl.ds(0, L)]
                for j in range(n_vreg):
                    buf[row0 + r, pl.ds(j * L, L)] = (
                        buf[row0 + r, pl.ds(j * L, L)] * (ws[j] * sb)
                    )

        gather(0, 0)
        gather(1, 1)

        def chunk_body(c, carry):
            slot = lax.rem(c, NBUF)
            nslot = lax.rem(c + 2, NBUF)
            wait_gather(slot)

            @pl.when(c >= 2)
            def _():
                wait_store(nslot)

            @pl.when(c + 2 < n_chunks)
            def _():
                gather(c + 2, nslot)

            normalize_chunk(slot)
            store(c, slot)
            return carry

        lax.fori_loop(0, n_chunks, chunk_body, 0)
        for c in (n_chunks - 2, n_chunks - 1):
            wait_store(lax.rem(jnp.int32(c), NBUF))

    return k


def kernel(input_ids, tok_embeddings, norm_weight):
    Bt, S = input_ids.shape
    V, D = tok_embeddings.shape
    ids = input_ids.reshape(-1)
    out = _make_fused(V, D, Bt * S)(ids, tok_embeddings, norm_weight)
    return out.reshape(Bt, S, D)


# speedup vs baseline: 5.5528x; 1.0136x over previous
import functools

import jax
import jax.numpy as jnp
from jax import lax
from jax.experimental import pallas as pl
from jax.experimental.pallas import tpu as pltpu
from jax.experimental.pallas import tpu_sc as plsc

_EPS = 1e-6


def _gather16(x, perm, L):
    dnums = lax.GatherDimensionNumbers(
        offset_dims=(), collapsed_slice_dims=(0,), start_index_map=(0,)
    )
    return lax.gather(
        x, perm.reshape(L, 1), dimension_numbers=dnums, slice_sizes=(1,),
        mode=lax.GatherScatterMode.PROMISE_IN_BOUNDS,
    )


def _vrsqrt(x):
    i = lax.bitcast_convert_type(x, jnp.int32)
    i = jnp.int32(0x5F3759DF) - lax.shift_right_arithmetic(i, 1)
    y = lax.bitcast_convert_type(i, jnp.float32)
    for _ in range(3):
        y = y * (1.5 - 0.5 * x * y * y)
    return y


@functools.cache
def _make_fused(V, D, B):
    info = plsc.get_sparse_core_info()
    NC, NS, L = info.num_cores, info.num_subcores, info.num_lanes
    NW = NC * NS
    n_vreg = D // L
    b_per_w = B // NW
    C = 64
    NBUF = 4
    n_chunks = b_per_w // C
    assert B % (8 * NW) == 0 and D % L == 0 and C % L == 0 and n_chunks >= 4

    mesh = plsc.VectorSubcoreMesh(core_axis_name="c", subcore_axis_name="s")

    @functools.partial(
        pl.kernel,
        mesh=mesh,
        out_type=jax.ShapeDtypeStruct((B, D), jnp.float32),
        scratch_types=[
            pltpu.VMEM((b_per_w,), jnp.int32),
            pltpu.VMEM((NBUF * C, D), jnp.float32),
            pltpu.VMEM((D,), jnp.float32),
            pltpu.VMEM((C, 16), jnp.float32),
            pltpu.SemaphoreType.DMA((NBUF,)),
            pltpu.SemaphoreType.DMA((NBUF,)),
        ],
    )
    def k(ids_hbm, table_hbm, w_hbm, out_hbm, idx_v, buf, wv, sc_v, gsem, ssem):
        wid = lax.axis_index("s") * NC + lax.axis_index("c")
        base = wid * b_per_w
        iota = jnp.arange(L, dtype=jnp.int32)
        masks = [jnp.asarray((iota & m) != 0) for m in (1, 2, 4, 8)]
        perms = [jnp.asarray(iota ^ m, dtype=jnp.int32) for m in (1, 2, 4, 8)]

        pltpu.sync_copy(w_hbm, wv)
        pltpu.sync_copy(ids_hbm.at[pl.ds(base, b_per_w)], idx_v)
        ws = [wv[pl.ds(j * L, L)] for j in range(n_vreg)]

        def bslice(slot):
            return buf.at[pl.ds(slot * C, C)]

        def gather(c, slot):
            pltpu.async_copy(
                table_hbm.at[idx_v.at[pl.ds(c * C, C)]], bslice(slot),
                gsem.at[slot],
            )

        def wait_gather(slot):
            pltpu.make_async_copy(
                table_hbm.at[pl.ds(0, C)], bslice(slot), gsem.at[slot]
            ).wait()

        def store(c, slot):
            pltpu.async_copy(
                bslice(slot), out_hbm.at[pl.ds(base + c * C, C)], ssem.at[slot]
            )

        def wait_store(slot):
            pltpu.make_async_copy(
                bslice(slot), out_hbm.at[pl.ds(base, C)], ssem.at[slot]
            ).wait()

        def combine(x, y, lvl):
            m = masks[lvl]
            return jnp.where(m, y, x) + _gather16(
                jnp.where(m, x, y), perms[lvl], L
            )

        def normalize_chunk(slot):
            row0 = slot * C

            @plsc.parallel_loop(0, C, unroll=4)
            def _(r):
                a0 = jnp.zeros((L,), jnp.float32)
                a1 = jnp.zeros((L,), jnp.float32)
                a2 = jnp.zeros((L,), jnp.float32)
                for j in range(0, n_vreg, 3):
                    v = buf[row0 + r, pl.ds(j * L, L)]
                    a0 = a0 + v * v
                    v = buf[row0 + r, pl.ds((j + 1) * L, L)]
                    a1 = a1 + v * v
                    v = buf[row0 + r, pl.ds((j + 2) * L, L)]
                    a2 = a2 + v * v
                acc = a0 + (a1 + a2)
                for lvl in range(4):
                    acc = acc + _gather16(acc, perms[lvl], L)
                sc_v[r, pl.ds(0, L)] = _vrsqrt(acc * (1.0 / D) + _EPS)

            @plsc.parallel_loop(0, C, unroll=4)
            def _(r):
                sb = sc_v[r, pl.ds(0, L)]
                for j in range(n_vreg):
                    buf[row0 + r, pl.ds(j * L, L)] = (
                        buf[row0 + r, pl.ds(j * L, L)] * (ws[j] * sb)
                    )

        gather(0, 0)
        gather(1, 1)

        def chunk_body(c, carry):
            slot = lax.rem(c, NBUF)
            nslot = lax.rem(c + 2, NBUF)
            wait_gather(slot)

            @pl.when(c >= 2)
            def _():
                wait_store(nslot)

            @pl.when(c + 2 < n_chunks)
            def _():
                gather(c + 2, nslot)

            normalize_chunk(slot)
            store(c, slot)
            return carry

        lax.fori_loop(0, n_chunks, chunk_body, 0)
        for c in (n_chunks - 2, n_chunks - 1):
            wait_store(lax.rem(jnp.int32(c), NBUF))

    return k


def kernel(input_ids, tok_embeddings, norm_weight):
    Bt, S = input_ids.shape
    V, D = tok_embeddings.shape
    ids = input_ids.reshape(-1)
    out = _make_fused(V, D, Bt * S)(ids, tok_embeddings, norm_weight)
    return out.reshape(Bt, S, D)


# speedup vs baseline: 5.5590x; 1.0011x over previous
import functools

import jax
import jax.numpy as jnp
from jax import lax
from jax.experimental import pallas as pl
from jax.experimental.pallas import tpu as pltpu
from jax.experimental.pallas import tpu_sc as plsc

_EPS = 1e-6


def _gather16(x, perm, L):
    dnums = lax.GatherDimensionNumbers(
        offset_dims=(), collapsed_slice_dims=(0,), start_index_map=(0,)
    )
    return lax.gather(
        x, perm.reshape(L, 1), dimension_numbers=dnums, slice_sizes=(1,),
        mode=lax.GatherScatterMode.PROMISE_IN_BOUNDS,
    )


def _vrsqrt(x):
    i = lax.bitcast_convert_type(x, jnp.int32)
    i = jnp.int32(0x5F3759DF) - lax.shift_right_arithmetic(i, 1)
    y = lax.bitcast_convert_type(i, jnp.float32)
    for _ in range(3):
        y = y * (1.5 - 0.5 * x * y * y)
    return y


@functools.cache
def _make_fused(V, D, B):
    info = plsc.get_sparse_core_info()
    NC, NS, L = info.num_cores, info.num_subcores, info.num_lanes
    NW = NC * NS
    n_vreg = D // L
    b_per_w = B // NW
    C = 64
    NBUF = 4
    n_chunks = b_per_w // C
    assert B % (8 * NW) == 0 and D % L == 0 and C % L == 0 and n_chunks >= 4

    mesh = plsc.VectorSubcoreMesh(core_axis_name="c", subcore_axis_name="s")

    @functools.partial(
        pl.kernel,
        mesh=mesh,
        out_type=jax.ShapeDtypeStruct((B, D), jnp.float32),
        scratch_types=[
            pltpu.VMEM((b_per_w,), jnp.int32),
            pltpu.VMEM((NBUF * C, D), jnp.float32),
            pltpu.VMEM((D,), jnp.float32),
            pltpu.VMEM((C, 16), jnp.float32),
            pltpu.SemaphoreType.DMA((NBUF,)),
            pltpu.SemaphoreType.DMA((NBUF,)),
        ],
    )
    def k(ids_hbm, table_hbm, w_hbm, out_hbm, idx_v, buf, wv, sc_v, gsem, ssem):
        wid = lax.axis_index("s") * NC + lax.axis_index("c")
        base = wid * b_per_w
        iota = jnp.arange(L, dtype=jnp.int32)
        perms = [jnp.asarray(iota ^ m, dtype=jnp.int32) for m in (1, 2, 4, 8)]

        pltpu.sync_copy(w_hbm, wv)
        pltpu.sync_copy(ids_hbm.at[pl.ds(base, b_per_w)], idx_v)
        ws = [wv[pl.ds(j * L, L)] for j in range(n_vreg)]

        def bslice(slot):
            return buf.at[pl.ds(slot * C, C)]

        def gather(c, slot):
            pltpu.async_copy(
                table_hbm.at[idx_v.at[pl.ds(c * C, C)]], bslice(slot),
                gsem.at[slot],
            )

        def wait_gather(slot):
            pltpu.make_async_copy(
                table_hbm.at[pl.ds(0, C)], bslice(slot), gsem.at[slot]
            ).wait()

        def store(c, slot):
            pltpu.async_copy(
                bslice(slot), out_hbm.at[pl.ds(base + c * C, C)], ssem.at[slot]
            )

        def wait_store(slot):
            pltpu.make_async_copy(
                bslice(slot), out_hbm.at[pl.ds(base, C)], ssem.at[slot]
            ).wait()

        def normalize_chunk(slot):
            row0 = slot * C

            @plsc.parallel_loop(0, C, unroll=4)
            def _(r):
                a0 = jnp.zeros((L,), jnp.float32)
                a1 = jnp.zeros((L,), jnp.float32)
                a2 = jnp.zeros((L,), jnp.float32)
                for j in range(0, n_vreg, 3):
                    v = buf[row0 + r, pl.ds(j * L, L)]
                    a0 = a0 + v * v
                    v = buf[row0 + r, pl.ds((j + 1) * L, L)]
                    a1 = a1 + v * v
                    v = buf[row0 + r, pl.ds((j + 2) * L, L)]
                    a2 = a2 + v * v
                acc = a0 + (a1 + a2)
                for lvl in range(4):
                    acc = acc + _gather16(acc, perms[lvl], L)
                sc_v[r, pl.ds(0, L)] = _vrsqrt(acc * (1.0 / D) + _EPS)

            @plsc.parallel_loop(0, C, unroll=4)
            def _(r):
                sb = sc_v[r, pl.ds(0, L)]
                for j in range(n_vreg):
                    buf[row0 + r, pl.ds(j * L, L)] = (
                        buf[row0 + r, pl.ds(j * L, L)] * (ws[j] * sb)
                    )

        gather(0, 0)
        gather(1, 1)

        def chunk_body(c, carry):
            slot = lax.rem(c, NBUF)
            nslot = lax.rem(c + 2, NBUF)
            wait_gather(slot)

            @pl.when(c >= 2)
            def _():
                wait_store(nslot)

            @pl.when(c + 2 < n_chunks)
            def _():
                gather(c + 2, nslot)

            normalize_chunk(slot)
            store(c, slot)
            return carry

        lax.fori_loop(0, n_chunks, chunk_body, 0)
        for c in (n_chunks - 2, n_chunks - 1):
            wait_store(lax.rem(jnp.int32(c), NBUF))

    return k


def kernel(input_ids, tok_embeddings, norm_weight):
    Bt, S = input_ids.shape
    V, D = tok_embeddings.shape
    ids = input_ids.reshape(-1)
    out = _make_fused(V, D, Bt * S)(ids, tok_embeddings, norm_weight)
    return out.reshape(Bt, S, D)
